# Initial kernel scaffold; baseline (speedup 1.0000x reference)
import jax
import jax.numpy as jnp
from jax.experimental import pallas as pl


def kernel(input, Codebook):
    raise NotImplementedError("write your pallas kernel here")


# speedup vs baseline: 1.2674x; 1.2674x over previous
import functools

import jax
import jax.numpy as jnp
from jax.experimental import pallas as pl
from jax.experimental.pallas import tpu as pltpu

SUB_LEVEL = 4
SUB_CENTERS = 1024
DIM = 256
BLOCK_N = 256


def _vq_kernel(x_ref, cb_ref, qh_ref, qs_ref, hc0_ref, hc1_ref, hc2_ref,
               hc3_ref, stats_ref):
    hc_refs = (hc0_ref, hc1_ref, hc2_ref, hc3_ref)
    pid = pl.program_id(0)

    @pl.when(pid == 0)
    def _init():
        for k in range(8):
            stats_ref[k] = 0.0

    x = x_ref[...]
    r = x
    qs = jnp.zeros_like(x)
    qh = jnp.zeros_like(x)
    sd_sum = 0.0
    hd_sum = 0.0
    hd_last = 0.0

    lane_iota = jax.lax.broadcasted_iota(jnp.int32, (BLOCK_N, SUB_CENTERS), 1)

    for level in range(SUB_LEVEL):
        c = cb_ref[level]
        b2 = jnp.sum(c * c, axis=1)[None, :]
        a2 = jnp.sum(r * r, axis=1, keepdims=True)
        g = jax.lax.dot_general(
            r, c, (((1,), (1,)), ((), ())),
            preferred_element_type=jnp.float32)
        diff = a2 + b2 - 2.0 * g
        maxi = jnp.max(diff, axis=1, keepdims=True)
        dist = -1.0 * (diff / maxi)
        p = jax.nn.softmax(dist, axis=1)
        soft = jax.lax.dot_general(
            p, c, (((1,), (0,)), ((), ())),
            preferred_element_type=jnp.float32)
        dmax = jnp.max(dist, axis=1, keepdims=True)
        is_max = dist == dmax
        code = jnp.min(jnp.where(is_max, lane_iota, SUB_CENTERS),
                       axis=1).astype(jnp.int32)
        onehot = (lane_iota == code[:, None]).astype(jnp.float32)
        hard = jax.lax.dot_general(
            onehot, c, (((1,), (0,)), ((), ())),
            precision=jax.lax.Precision.HIGHEST,
            preferred_element_type=jnp.float32)
        r = r - hard
        qs = qs + soft
        qh = qh + hard
        t = x - qs
        sd_sum = sd_sum + jnp.sum(t * t)
        u = x - qh
        hd_block = jnp.sum(u * u)
        hd_sum = hd_sum + hd_block
        if level == SUB_LEVEL - 1:
            hd_last = hd_block
        hc_refs[level][...] = code[:, None]

    qh_ref[...] = qh
    qs_ref[...] = qs
    d = qs - qh
    jc_sum = jnp.sum(d * d)

    stats_ref[0] += sd_sum
    stats_ref[1] += hd_sum
    stats_ref[2] += hd_last
    stats_ref[3] += jc_sum


@jax.jit
def kernel(input, Codebook):
    n, d = input.shape
    num_blocks = n // BLOCK_N
    grid = (num_blocks,)
    out_shape = (
        jax.ShapeDtypeStruct((n, d), jnp.float32),
        jax.ShapeDtypeStruct((n, d), jnp.float32),
        jax.ShapeDtypeStruct((n, 1), jnp.int32),
        jax.ShapeDtypeStruct((n, 1), jnp.int32),
        jax.ShapeDtypeStruct((n, 1), jnp.int32),
        jax.ShapeDtypeStruct((n, 1), jnp.int32),
        jax.ShapeDtypeStruct((8,), jnp.float32),
    )
    row_spec = pl.BlockSpec((BLOCK_N, d), lambda i: (i, 0))
    code_spec = pl.BlockSpec((BLOCK_N, 1), lambda i: (i, 0))
    out = pl.pallas_call(
        _vq_kernel,
        grid=grid,
        in_specs=[
            row_spec,
            pl.BlockSpec((SUB_LEVEL, SUB_CENTERS, DIM), lambda i: (0, 0, 0)),
        ],
        out_specs=(
            row_spec,
            row_spec,
            code_spec,
            code_spec,
            code_spec,
            code_spec,
            pl.BlockSpec(memory_space=pltpu.SMEM),
        ),
        out_shape=out_shape,
    )(input, Codebook)
    qhard, qsoft, hc0, hc1, hc2, hc3, stats = out
    nf = jnp.float32(n)
    soft_distortion = stats[0] / nf
    hard_distortion = stats[1] / nf
    error = stats[2] / nf
    joint_center = stats[3] / (nf * jnp.float32(d))
    hard_code = jnp.concatenate([hc0, hc1, hc2, hc3], axis=1)
    return (qhard, qsoft, soft_distortion, hard_distortion, joint_center,
            error, hard_code)


# speedup vs baseline: 1.8168x; 1.4334x over previous
import functools

import jax
import jax.numpy as jnp
from jax.experimental import pallas as pl
from jax.experimental.pallas import tpu as pltpu

SUB_LEVEL = 4
SUB_CENTERS = 1024
DIM = 256
BLOCK_N = 256


def _vq_kernel(x_ref, cb_ref, qh_ref, qs_ref, hc0_ref, hc1_ref, hc2_ref,
               hc3_ref, stats_ref):
    hc_refs = (hc0_ref, hc1_ref, hc2_ref, hc3_ref)
    pid = pl.program_id(0)

    @pl.when(pid == 0)
    def _init():
        for k in range(8):
            stats_ref[k] = 0.0

    x = x_ref[...]
    r = x
    qs = jnp.zeros_like(x)
    qh = jnp.zeros_like(x)
    sd_sum = 0.0
    hd_sum = 0.0
    hd_last = 0.0

    lane_iota = jax.lax.broadcasted_iota(jnp.int32, (BLOCK_N, SUB_CENTERS), 1)

    for level in range(SUB_LEVEL):
        c = cb_ref[level]
        h1 = c.astype(jnp.bfloat16)
        r1 = c - h1.astype(jnp.float32)
        h2 = r1.astype(jnp.bfloat16)
        h3 = (r1 - h2.astype(jnp.float32)).astype(jnp.bfloat16)
        b2 = jnp.sum(c * c, axis=1)[None, :]
        a2 = jnp.sum(r * r, axis=1, keepdims=True)
        g = jax.lax.dot_general(
            r, c, (((1,), (1,)), ((), ())),
            preferred_element_type=jnp.float32)
        diff = a2 + b2 - 2.0 * g
        maxi = jnp.max(diff, axis=1, keepdims=True)
        q = diff / maxi
        qmin = jnp.min(q, axis=1, keepdims=True)
        e = jnp.exp(qmin - q)
        p = e / jnp.sum(e, axis=1, keepdims=True)
        soft = jax.lax.dot_general(
            p, c, (((1,), (0,)), ((), ())),
            preferred_element_type=jnp.float32)
        code = jnp.min(jnp.where(q == qmin, lane_iota, SUB_CENTERS),
                       axis=1).astype(jnp.int32)
        onehot = (lane_iota == code[:, None]).astype(jnp.bfloat16)
        hard = jax.lax.dot_general(
            onehot, h1, (((1,), (0,)), ((), ())),
            preferred_element_type=jnp.float32)
        hard = hard + jax.lax.dot_general(
            onehot, h2, (((1,), (0,)), ((), ())),
            preferred_element_type=jnp.float32)
        hard = hard + jax.lax.dot_general(
            onehot, h3, (((1,), (0,)), ((), ())),
            preferred_element_type=jnp.float32)
        r = r - hard
        qs = qs + soft
        qh = qh + hard
        t = x - qs
        sd_sum = sd_sum + jnp.sum(t * t)
        u = x - qh
        hd_block = jnp.sum(u * u)
        hd_sum = hd_sum + hd_block
        if level == SUB_LEVEL - 1:
            hd_last = hd_block
        hc_refs[level][...] = code[:, None]

    qh_ref[...] = qh
    qs_ref[...] = qs
    d = qs - qh
    jc_sum = jnp.sum(d * d)

    stats_ref[0] += sd_sum
    stats_ref[1] += hd_sum
    stats_ref[2] += hd_last
    stats_ref[3] += jc_sum


@jax.jit
def kernel(input, Codebook):
    n, d = input.shape
    num_blocks = n // BLOCK_N
    grid = (num_blocks,)
    out_shape = (
        jax.ShapeDtypeStruct((n, d), jnp.float32),
        jax.ShapeDtypeStruct((n, d), jnp.float32),
        jax.ShapeDtypeStruct((n, 1), jnp.int32),
        jax.ShapeDtypeStruct((n, 1), jnp.int32),
        jax.ShapeDtypeStruct((n, 1), jnp.int32),
        jax.ShapeDtypeStruct((n, 1), jnp.int32),
        jax.ShapeDtypeStruct((8,), jnp.float32),
    )
    row_spec = pl.BlockSpec((BLOCK_N, d), lambda i: (i, 0))
    code_spec = pl.BlockSpec((BLOCK_N, 1), lambda i: (i, 0))
    out = pl.pallas_call(
        _vq_kernel,
        grid=grid,
        in_specs=[
            row_spec,
            pl.BlockSpec((SUB_LEVEL, SUB_CENTERS, DIM), lambda i: (0, 0, 0)),
        ],
        out_specs=(
            row_spec,
            row_spec,
            code_spec,
            code_spec,
            code_spec,
            code_spec,
            pl.BlockSpec(memory_space=pltpu.SMEM),
        ),
        out_shape=out_shape,
    )(input, Codebook)
    qhard, qsoft, hc0, hc1, hc2, hc3, stats = out
    nf = jnp.float32(n)
    soft_distortion = stats[0] / nf
    hard_distortion = stats[1] / nf
    error = stats[2] / nf
    joint_center = stats[3] / (nf * jnp.float32(d))
    hard_code = jnp.concatenate([hc0, hc1, hc2, hc3], axis=1)
    return (qhard, qsoft, soft_distortion, hard_distortion, joint_center,
            error, hard_code)


# speedup vs baseline: 2.0490x; 1.1278x over previous
import jax
import jax.numpy as jnp
from jax.experimental import pallas as pl
from jax.experimental.pallas import tpu as pltpu

SUB_LEVEL = 4
SUB_CENTERS = 1024
DIM = 256
BLOCK_N = 512


def _vq_kernel(x_ref, cb_ref, qh_ref, qs_ref, hc0_ref, hc1_ref, hc2_ref,
               hc3_ref, stats_ref, h1_s, h2_s, h3_s, b2_s):
    hc_refs = (hc0_ref, hc1_ref, hc2_ref, hc3_ref)
    pid = pl.program_id(0)

    @pl.when(pid == 0)
    def _init():
        for k in range(8):
            stats_ref[k] = 0.0
        for level in range(SUB_LEVEL):
            c = cb_ref[level]
            h1 = c.astype(jnp.bfloat16)
            r1 = c - h1.astype(jnp.float32)
            h2 = r1.astype(jnp.bfloat16)
            h3 = (r1 - h2.astype(jnp.float32)).astype(jnp.bfloat16)
            h1_s[level] = h1
            h2_s[level] = h2
            h3_s[level] = h3
            b2_s[level] = jnp.sum(c * c, axis=1)

    x = x_ref[...]
    r = x
    qs = jnp.zeros_like(x)
    qh = jnp.zeros_like(x)
    sd_sum = 0.0
    hd_sum = 0.0
    hd_last = 0.0

    lane_iota = jax.lax.broadcasted_iota(jnp.int32, (BLOCK_N, SUB_CENTERS), 1)

    for level in range(SUB_LEVEL):
        c = cb_ref[level]
        b2 = b2_s[level][None, :]
        a2 = jnp.sum(r * r, axis=1, keepdims=True)
        g = jax.lax.dot_general(
            r, c, (((1,), (1,)), ((), ())),
            preferred_element_type=jnp.float32)
        diff = a2 + b2 - 2.0 * g
        maxi = jnp.max(diff, axis=1, keepdims=True)
        q = diff / maxi
        qmin = jnp.min(q, axis=1, keepdims=True)
        e = jnp.exp(qmin - q)
        p = e / jnp.sum(e, axis=1, keepdims=True)
        soft = jax.lax.dot_general(
            p, c, (((1,), (0,)), ((), ())),
            preferred_element_type=jnp.float32)
        code = jnp.min(jnp.where(q == qmin, lane_iota, SUB_CENTERS),
                       axis=1).astype(jnp.int32)
        onehot = (lane_iota == code[:, None]).astype(jnp.bfloat16)
        hard = jax.lax.dot_general(
            onehot, h1_s[level], (((1,), (0,)), ((), ())),
            preferred_element_type=jnp.float32)
        hard = hard + jax.lax.dot_general(
            onehot, h2_s[level], (((1,), (0,)), ((), ())),
            preferred_element_type=jnp.float32)
        hard = hard + jax.lax.dot_general(
            onehot, h3_s[level], (((1,), (0,)), ((), ())),
            preferred_element_type=jnp.float32)
        r = r - hard
        qs = qs + soft
        qh = qh + hard
        t = x - qs
        sd_sum = sd_sum + jnp.sum(t * t)
        u = x - qh
        hd_block = jnp.sum(u * u)
        hd_sum = hd_sum + hd_block
        if level == SUB_LEVEL - 1:
            hd_last = hd_block
        hc_refs[level][...] = code[:, None]

    qh_ref[...] = qh
    qs_ref[...] = qs
    d = qs - qh
    jc_sum = jnp.sum(d * d)

    stats_ref[0] += sd_sum
    stats_ref[1] += hd_sum
    stats_ref[2] += hd_last
    stats_ref[3] += jc_sum


@jax.jit
def kernel(input, Codebook):
    n, d = input.shape
    num_blocks = n // BLOCK_N
    grid = (num_blocks,)
    out_shape = (
        jax.ShapeDtypeStruct((n, d), jnp.float32),
        jax.ShapeDtypeStruct((n, d), jnp.float32),
        jax.ShapeDtypeStruct((n, 1), jnp.int32),
        jax.ShapeDtypeStruct((n, 1), jnp.int32),
        jax.ShapeDtypeStruct((n, 1), jnp.int32),
        jax.ShapeDtypeStruct((n, 1), jnp.int32),
        jax.ShapeDtypeStruct((8,), jnp.float32),
    )
    row_spec = pl.BlockSpec((BLOCK_N, d), lambda i: (i, 0))
    code_spec = pl.BlockSpec((BLOCK_N, 1), lambda i: (i, 0))
    out = pl.pallas_call(
        _vq_kernel,
        grid=grid,
        in_specs=[
            row_spec,
            pl.BlockSpec((SUB_LEVEL, SUB_CENTERS, DIM), lambda i: (0, 0, 0)),
        ],
        out_specs=(
            row_spec,
            row_spec,
            code_spec,
            code_spec,
            code_spec,
            code_spec,
            pl.BlockSpec(memory_space=pltpu.SMEM),
        ),
        out_shape=out_shape,
        scratch_shapes=[
            pltpu.VMEM((SUB_LEVEL, SUB_CENTERS, DIM), jnp.bfloat16),
            pltpu.VMEM((SUB_LEVEL, SUB_CENTERS, DIM), jnp.bfloat16),
            pltpu.VMEM((SUB_LEVEL, SUB_CENTERS, DIM), jnp.bfloat16),
            pltpu.VMEM((SUB_LEVEL, SUB_CENTERS), jnp.float32),
        ],
    )(input, Codebook)
    qhard, qsoft, hc0, hc1, hc2, hc3, stats = out
    nf = jnp.float32(n)
    soft_distortion = stats[0] / nf
    hard_distortion = stats[1] / nf
    error = stats[2] / nf
    joint_center = stats[3] / (nf * jnp.float32(d))
    hard_code = jnp.concatenate([hc0, hc1, hc2, hc3], axis=1)
    return (qhard, qsoft, soft_distortion, hard_distortion, joint_center,
            error, hard_code)
